# Initial kernel scaffold; baseline (speedup 1.0000x reference)
import jax
import jax.numpy as jnp
from jax.experimental import pallas as pl


def kernel(x, table):
    raise NotImplementedError("write your pallas kernel here")


# speedup vs baseline: 9.1817x; 9.1817x over previous
import functools
import math

import jax
import jax.numpy as jnp
from jax import lax
from jax.experimental import pallas as pl
from jax.experimental.pallas import tpu as pltpu
from jax.experimental.pallas import tpu_sc as plsc

D_MODEL = 128
SCALE = math.sqrt(128.0)
LANES = 16

C = 256
IPR = 128
K = C // IPR


def _sc_gather_scale(flat_idx2d, table, *, num_workers, b_per_w):
    n_chunks = b_per_w // C
    rows_pw = b_per_w // IPR
    total = num_workers * b_per_w

    mesh = plsc.VectorSubcoreMesh(core_axis_name="c", subcore_axis_name="s")

    @functools.partial(
        pl.kernel,
        mesh=mesh,
        out_type=jax.ShapeDtypeStruct((total, D_MODEL), jnp.float32),
        scratch_types=[
            pltpu.VMEM((K, IPR), jnp.int32),
            pltpu.VMEM((K, IPR), jnp.int32),
            pltpu.VMEM((C, D_MODEL), jnp.float32),
            pltpu.VMEM((C, D_MODEL), jnp.float32),
            pltpu.SemaphoreType.DMA,
            pltpu.SemaphoreType.DMA,
            pltpu.SemaphoreType.DMA,
            pltpu.SemaphoreType.DMA,
        ],
    )
    def body(idx_hbm, table_hbm, out_hbm, idx0, idx1, rows0, rows1,
             g0, g1, p0, p1):
        nc = 2
        wid = lax.axis_index("s") * nc + lax.axis_index("c")
        idx_row0 = wid * rows_pw
        out_row0 = wid * b_per_w

        idx_bufs = (idx0, idx1)
        rows_bufs = (rows0, rows1)
        g_sems = (g0, g1)
        p_sems = (p0, p1)

        def load_idx(i, buf):
            pltpu.sync_copy(idx_hbm.at[pl.ds(idx_row0 + i * K, K)],
                            idx_bufs[buf])

        def gather_descs(buf):
            return [
                pltpu.make_async_copy(
                    table_hbm.at[idx_bufs[buf].at[k]],
                    rows_bufs[buf].at[pl.ds(k * IPR, IPR)],
                    g_sems[buf],
                )
                for k in range(K)
            ]

        def put_desc(i, buf):
            return pltpu.make_async_copy(
                rows_bufs[buf],
                out_hbm.at[pl.ds(out_row0 + i * C, C)],
                p_sems[buf],
            )

        def scale_buf(buf):
            rows = rows_bufs[buf]

            def srow(r, carry):
                for l in range(D_MODEL // LANES):
                    sl = pl.ds(l * LANES, LANES)
                    rows[r, sl] = rows[r, sl] * SCALE
                return carry

            lax.fori_loop(0, C, srow, 0, unroll=2)

        load_idx(0, 0)
        for d in gather_descs(0):
            d.start()
        load_idx(1, 1)
        for d in gather_descs(1):
            d.start()

        def step(i, buf):
            for d in gather_descs(buf):
                d.wait()
            scale_buf(buf)
            put_desc(i, buf).start()
            load_idx(i + 2, buf)
            put_desc(i, buf).wait()
            for d in gather_descs(buf):
                d.start()

        def pair(j, carry):
            step(2 * j, 0)
            step(2 * j + 1, 1)
            return carry

        lax.fori_loop(0, (n_chunks - 2) // 2, pair, 0)

        for i, buf in ((n_chunks - 2, 0), (n_chunks - 1, 1)):
            for d in gather_descs(buf):
                d.wait()
            scale_buf(buf)
            put_desc(i, buf).start()
        for i, buf in ((n_chunks - 2, 0), (n_chunks - 1, 1)):
            put_desc(i, buf).wait()

    return body(flat_idx2d, table)


def kernel(x, table):
    b, s = x.shape
    total = b * s
    num_workers = 32
    b_per_w = total // num_workers
    flat2d = x.reshape(total // IPR, IPR).astype(jnp.int32)
    out = _sc_gather_scale(flat2d, table,
                           num_workers=num_workers, b_per_w=b_per_w)
    return out.reshape(b, s, D_MODEL)
